# baseline (speedup 1.0000x reference)
import jax
import jax.numpy as jnp
from jax.experimental import pallas as pl
from jax.experimental.pallas import tpu as pltpu


_L = 128
_NG = 32
_SS = 8


def _gather_body(idx_ref, col_ref, o_ref):
    g = pl.program_id(0)
    j = pl.program_id(1)
    r = idx_ref[g * _L + j] % _L
    lane = jax.lax.broadcasted_iota(jnp.int32, (64, _L), 1)
    e = jnp.sum(jnp.where(lane == r, col_ref[...], 0.0), axis=1, keepdims=True)

    @pl.when(j == 0)
    def _():
        o_ref[...] = jnp.zeros_like(o_ref)

    o_ref[0] = jnp.where(lane == j, e, o_ref[0])


def _add_body(x_ref, e_ref, o_ref):
    o_ref[...] = x_ref[...] + e_ref[0]


def kernel(x, idxs, embedding_weight):
    B, S, D = x.shape
    xv = jnp.transpose(x, (1, 2, 0))
    tt = embedding_weight.T

    emb_t = pl.pallas_call(
        _gather_body,
        grid_spec=pltpu.PrefetchScalarGridSpec(
            num_scalar_prefetch=1,
            grid=(_NG, _L),
            in_specs=[
                pl.BlockSpec((D, _L), lambda g, j, idx: (0, idx[g * _L + j] // _L)),
            ],
            out_specs=pl.BlockSpec((1, D, _L), lambda g, j, idx: (g, 0, 0)),
        ),
        out_shape=jax.ShapeDtypeStruct((_NG, D, _L), jnp.float32),
    )(idxs, tt)

    outv = pl.pallas_call(
        _add_body,
        grid=(_NG, S // _SS),
        in_specs=[
            pl.BlockSpec((_SS, D, _L), lambda g, i: (i, 0, g)),
            pl.BlockSpec((1, D, _L), lambda g, i: (g, 0, 0)),
        ],
        out_specs=pl.BlockSpec((_SS, D, _L), lambda g, i: (i, 0, g)),
        out_shape=jax.ShapeDtypeStruct((S, D, B), jnp.float32),
    )(xv, emb_t)
    return jnp.transpose(outv, (2, 0, 1))


# speedup vs baseline: 11.5152x; 11.5152x over previous
import functools

import jax
import jax.numpy as jnp
from jax import lax
from jax.experimental import pallas as pl
from jax.experimental.pallas import tpu as pltpu
from jax.experimental.pallas import tpu_sc as plsc

_NC = 2
_NS = 16
_NW = _NC * _NS
_L = 128
_NBUF = 4
_SS = 8


def _sc_gather(tt, idxs):
    D, V = tt.shape
    B = idxs.shape[0]
    perw = B // _NW
    mesh = plsc.VectorSubcoreMesh(core_axis_name="c", subcore_axis_name="s")

    @functools.partial(
        pl.kernel,
        mesh=mesh,
        out_type=jax.ShapeDtypeStruct((D, B), jnp.float32),
        scratch_types=[
            pltpu.VMEM((perw + 16,), jnp.int32),
            pltpu.VMEM((_NBUF, D, _L), jnp.float32),
            pltpu.VMEM((D, perw), jnp.float32),
            pltpu.SemaphoreType.DMA((_NBUF,)),
        ],
        compiler_params=pltpu.CompilerParams(needs_layout_passes=False),
    )
    def gather_kernel(tt_hbm, idx_hbm, out_hbm, idx_v, colbuf, outbuf, sems):
        wid = lax.axis_index("s") * _NC + lax.axis_index("c")
        base = wid * perw
        pltpu.sync_copy(idx_hbm.at[pl.ds(base, perw)], idx_v.at[pl.ds(0, perw)])

        def idx_at(j):
            return idx_v[pl.ds(j, 16)][0]

        def start_fetch(j):
            q = pl.multiple_of((idx_at(j) // _L) * _L, _L)
            pltpu.make_async_copy(
                tt_hbm.at[:, pl.ds(q, _L)], colbuf.at[j % _NBUF], sems.at[j % _NBUF]
            ).start()

        for j in range(_NBUF):
            start_fetch(j)

        iota16 = lax.iota(jnp.int32, 16)

        def body(j, carry):
            pltpu.make_async_copy(
                tt_hbm.at[:, pl.ds(0, _L)], colbuf.at[j % _NBUF], sems.at[j % _NBUF]
            ).wait()
            rv = jnp.full((16,), idx_at(j) % _L, jnp.int32)
            jv = jnp.full((16,), j, jnp.int32)
            tile = colbuf.at[j % _NBUF]
            for c in range(D // 16):
                dv = iota16 + 16 * c
                vals = plsc.load_gather(tile, [dv, rv])
                plsc.store_scatter(outbuf, [dv, jv], vals)

            @pl.when(j + _NBUF < perw)
            def _():
                start_fetch(j + _NBUF)

            return carry

        lax.fori_loop(0, perw, body, 0, unroll=False)
        pltpu.sync_copy(outbuf, out_hbm.at[:, pl.ds(base, perw)])

    return gather_kernel(tt, idxs)


def _add_body(x_ref, e_ref, o_ref):
    o_ref[...] = x_ref[...] + e_ref[...]


def kernel(x, idxs, embedding_weight):
    B, S, D = x.shape
    xv = jnp.transpose(x, (1, 2, 0))
    tt = embedding_weight.T

    emb_t = _sc_gather(tt, idxs)

    outv = pl.pallas_call(
        _add_body,
        grid=(S // _SS,),
        in_specs=[
            pl.BlockSpec((_SS, D, B), lambda i: (i, 0, 0)),
            pl.BlockSpec((D, B), lambda i: (0, 0)),
        ],
        out_specs=pl.BlockSpec((_SS, D, B), lambda i: (i, 0, 0)),
        out_shape=jax.ShapeDtypeStruct((S, D, B), jnp.float32),
    )(xv, emb_t)
    return jnp.transpose(outv, (2, 0, 1))
